# baseline (device time: 44382 ns/iter reference)
import jax
import jax.numpy as jnp
from jax import lax
from jax.experimental import pallas as pl
from jax.experimental.pallas import tpu as pltpu

N_DEV = 16
NS = 4

N_C = 7


def _cidx(kind, d):
    return d if kind == "bel" else 3 + d


def _gelu_f32(y):
    c = 0.7978845608028654
    return 0.5 * y * (1.0 + jnp.tanh(c * (y + 0.044715 * y * y * y)))


def kernel(x, w_mat):
    m_per, k = x.shape
    _, n_per = w_mat.shape
    ms = m_per // NS

    def body(x_ref, w_ref, out_ref, comm_ref, w_bf,
             csend, crecv, ssend, srecv):
        my = lax.axis_index("i")
        z = my // 4
        q = my % 4
        up = my + 4
        down = my - 4
        right = 4 * z + (q + 1) % 4
        left = 4 * z + (q + 3) % 4

        has_up = z < 3
        has_dn = z > 0

        def pred_c(c):
            if c == 0:
                return None
            if c <= 3:
                return z >= c
            return z + (c - 3) <= 3

        dq = {"me": 0, "L": 3, "R": 1, "D": 2}

        def origin(kind, c):
            qq = (q + dq[kind]) % 4
            if c == 0:
                return 4 * z + qq
            if c <= 3:
                return 4 * (z - c) + qq
            return 4 * (z + (c - 3)) + qq

        barrier_sem = pltpu.get_barrier_semaphore()
        for nbr in (left, right):
            pl.semaphore_signal(barrier_sem, inc=1, device_id=(nbr,),
                                device_id_type=pl.DeviceIdType.MESH)

        @pl.when(has_up)
        def _():
            pl.semaphore_signal(barrier_sem, inc=1, device_id=(up,),
                                device_id_type=pl.DeviceIdType.MESH)

        @pl.when(has_dn)
        def _():
            pl.semaphore_signal(barrier_sem, inc=1, device_id=(down,),
                                device_id_type=pl.DeviceIdType.MESH)

        pl.semaphore_wait(barrier_sem, 2)

        @pl.when(has_up)
        def _():
            pl.semaphore_wait(barrier_sem, 1)

        @pl.when(has_dn)
        def _():
            pl.semaphore_wait(barrier_sem, 1)

        comm_ref[0, :, :] = x_ref[:, :].astype(jnp.bfloat16)

        sends = []

        def _rdma(src_slot, dst_slot, dev, ssem, rsem, h):
            rows = pl.ds(h * ms, ms)
            return pltpu.make_async_remote_copy(
                src_ref=comm_ref.at[src_slot, rows],
                dst_ref=comm_ref.at[dst_slot, rows],
                send_sem=ssem, recv_sem=rsem,
                device_id=(dev,), device_id_type=pl.DeviceIdType.MESH,
            )

        def _start(rdma, pred):
            if pred is None:
                rdma.start()
            else:
                @pl.when(pred)
                def _():
                    rdma.start()
            sends.append((rdma, pred))

        def _wait(rdma, pred):
            if pred is None:
                rdma.wait_recv()
            else:
                @pl.when(pred)
                def _():
                    rdma.wait_recv()


        for h in range(NS):
            _start(_rdma(0, 1, up, csend.at[0, h], crecv.at[0, h], h),
                   has_up)
            _start(_rdma(0, 4, down, csend.at[1, h], crecv.at[3, h], h),
                   has_dn)
            _start(_rdma(0, 7, right, ssend.at[0, h], srecv.at[0, h], h),
                   None)
            _start(_rdma(0, 14, left, ssend.at[0, 4 + h],
                         srecv.at[0, 4 + h], h), None)

        w_bf[:, :] = w_ref[:, :].astype(jnp.bfloat16)
        y0 = jnp.dot(comm_ref[0, :, :], w_bf[:, :],
                     preferred_element_type=jnp.float32)
        out_ref[pl.ds(my * m_per, m_per), :] = _gelu_f32(y0)

        def _gemm_block(slot, kind, c, p):
            def _g():
                y = jnp.dot(comm_ref[slot, :, :], w_bf[:, :],
                            preferred_element_type=jnp.float32)
                out_ref[pl.ds(origin(kind, c) * m_per, m_per), :] = \
                    _gelu_f32(y)
            if p is None:
                _g()
            else:
                @pl.when(p)
                def _():
                    _g()

        def col_step(d):
            for kind in ("bel", "abv"):
                c = _cidx(kind, d)
                p = pred_c(c)
                isem = c - 1 if kind == "bel" else 3 + (c - 4)
                for h in range(NS):
                    _wait(_rdma(c, c, left, csend.at[0, h],
                                crecv.at[isem, h], h), p)
                    if d < 3:
                        nc = c + 1
                        if kind == "bel":
                            fp = jnp.logical_and(p, has_up)
                            _start(_rdma(c, nc, up, csend.at[2 + (d - 1), h],
                                         crecv.at[nc - 1, h], h), fp)
                        else:
                            fp = jnp.logical_and(p, has_dn)
                            _start(_rdma(c, nc, down,
                                         csend.at[4 + (d - 1), h],
                                         crecv.at[3 + (nc - 4), h], h), fp)
                    _start(_rdma(c, 7 + c, right, ssend.at[c, h],
                                 srecv.at[c, h], h), p)
                    _start(_rdma(c, 14 + c, left, ssend.at[c, 4 + h],
                                 srecv.at[c, 4 + h], h), p)
                _gemm_block(c, "me", c, p)

        def sq_step(c):
            p = pred_c(c)
            for h in (0, 1):
                _wait(_rdma(7 + c, 7 + c, left, ssend.at[c, h],
                            srecv.at[c, h], h), p)
                _start(_rdma(7 + c, 21 + c, right, ssend.at[c, 8 + h],
                             srecv.at[c, 8 + h], h), p)
            for h in (2, 3):
                _wait(_rdma(14 + c, 14 + c, left, ssend.at[c, 4 + h],
                            srecv.at[c, 4 + h], h), p)
                _start(_rdma(14 + c, 21 + c, left, ssend.at[c, 8 + h],
                             srecv.at[c, 8 + h], h), p)
            for h in (2, 3):
                _wait(_rdma(7 + c, 7 + c, left, ssend.at[c, h],
                            srecv.at[c, h], h), p)
            for h in (0, 1):
                _wait(_rdma(14 + c, 14 + c, left, ssend.at[c, 4 + h],
                            srecv.at[c, 4 + h], h), p)
            _gemm_block(7 + c, "L", c, p)
            _gemm_block(14 + c, "R", c, p)

        def diag_step(c):
            p = pred_c(c)
            for h in range(NS):
                _wait(_rdma(21 + c, 21 + c, left, ssend.at[c, 8 + h],
                            srecv.at[c, 8 + h], h), p)
            _gemm_block(21 + c, "D", c, p)

        col_step(1)
        sq_step(0)
        col_step(2)
        sq_step(1)
        sq_step(4)
        col_step(3)
        sq_step(2)
        sq_step(5)
        diag_step(0)
        sq_step(3)
        sq_step(6)
        diag_step(1)
        diag_step(4)
        diag_step(2)
        diag_step(5)
        diag_step(3)
        diag_step(6)

        for rdma, pred in sends:
            if pred is None:
                rdma.wait_send()
            else:
                @pl.when(pred)
                def _():
                    rdma.wait_send()

    return pl.pallas_call(
        body,
        out_shape=jax.ShapeDtypeStruct((N_DEV * m_per, n_per), jnp.float32),
        in_specs=[
            pl.BlockSpec(memory_space=pltpu.VMEM),
            pl.BlockSpec(memory_space=pltpu.VMEM),
        ],
        out_specs=pl.BlockSpec(memory_space=pltpu.VMEM),
        scratch_shapes=[
            pltpu.VMEM((28, m_per, k), jnp.bfloat16),
            pltpu.VMEM((k, n_per), jnp.bfloat16),
            pltpu.SemaphoreType.DMA((6, NS)),
            pltpu.SemaphoreType.DMA((6, NS)),
            pltpu.SemaphoreType.DMA((N_C, 12)),
            pltpu.SemaphoreType.DMA((N_C, 12)),
        ],
        compiler_params=pltpu.CompilerParams(collective_id=0),
    )(x, w_mat)


# device time: 43359 ns/iter; 1.0236x vs baseline; 1.0236x over previous
import jax
import jax.numpy as jnp
from jax import lax
from jax.experimental import pallas as pl
from jax.experimental.pallas import tpu as pltpu

N_DEV = 16

N_C = 7


def _cidx(kind, d):
    return d if kind == "bel" else 3 + d


def _gelu_f32(y):
    c = 0.7978845608028654
    return 0.5 * y * (1.0 + jnp.tanh(c * (y + 0.044715 * y * y * y)))


def kernel(x, w_mat):
    m_per, k = x.shape
    _, n_per = w_mat.shape
    mh = m_per // 2

    def body(x_ref, w_ref, out_ref, comm_ref, w_bf,
             csend, crecv, ssend, srecv):
        my = lax.axis_index("i")
        z = my // 4
        q = my % 4
        up = my + 4
        down = my - 4
        right = 4 * z + (q + 1) % 4
        left = 4 * z + (q + 3) % 4

        has_up = z < 3
        has_dn = z > 0

        def pred_c(c):
            if c == 0:
                return None
            if c <= 3:
                return z >= c
            return z + (c - 3) <= 3

        dq = {"me": 0, "L": 3, "R": 1, "D": 2}

        def origin(kind, c):
            qq = (q + dq[kind]) % 4
            if c == 0:
                return 4 * z + qq
            if c <= 3:
                return 4 * (z - c) + qq
            return 4 * (z + (c - 3)) + qq

        barrier_sem = pltpu.get_barrier_semaphore()
        for nbr in (left, right):
            pl.semaphore_signal(barrier_sem, inc=1, device_id=(nbr,),
                                device_id_type=pl.DeviceIdType.MESH)

        @pl.when(has_up)
        def _():
            pl.semaphore_signal(barrier_sem, inc=1, device_id=(up,),
                                device_id_type=pl.DeviceIdType.MESH)

        @pl.when(has_dn)
        def _():
            pl.semaphore_signal(barrier_sem, inc=1, device_id=(down,),
                                device_id_type=pl.DeviceIdType.MESH)

        pl.semaphore_wait(barrier_sem, 2)

        @pl.when(has_up)
        def _():
            pl.semaphore_wait(barrier_sem, 1)

        @pl.when(has_dn)
        def _():
            pl.semaphore_wait(barrier_sem, 1)

        comm_ref[0, :, :] = x_ref[:, :].astype(jnp.bfloat16)

        sends = []

        def _rdma(src_slot, dst_slot, dev, ssem, rsem, h):
            rows = pl.ds(h * mh, mh)
            return pltpu.make_async_remote_copy(
                src_ref=comm_ref.at[src_slot, rows],
                dst_ref=comm_ref.at[dst_slot, rows],
                send_sem=ssem, recv_sem=rsem,
                device_id=(dev,), device_id_type=pl.DeviceIdType.MESH,
            )

        def _start(rdma, pred):
            if pred is None:
                rdma.start()
            else:
                @pl.when(pred)
                def _():
                    rdma.start()
            sends.append((rdma, pred))

        def _wait(rdma, pred):
            if pred is None:
                rdma.wait_recv()
            else:
                @pl.when(pred)
                def _():
                    rdma.wait_recv()


        for h in (0, 1):
            _start(_rdma(0, 1, up, csend.at[0, h], crecv.at[0, h], h),
                   has_up)
            _start(_rdma(0, 4, down, csend.at[1, h], crecv.at[3, h], h),
                   has_dn)
            _start(_rdma(0, 7, right, ssend.at[0, h], srecv.at[0, h], h),
                   None)
            _start(_rdma(0, 14, left, ssend.at[0, 2 + h],
                         srecv.at[0, 2 + h], h), None)

        w_bf[:, :] = w_ref[:, :].astype(jnp.bfloat16)
        y0 = jnp.dot(comm_ref[0, :, :], w_bf[:, :],
                     preferred_element_type=jnp.float32)
        out_ref[pl.ds(my * m_per, m_per), :] = _gelu_f32(y0)

        def _gemm_block(slot, kind, c, p):
            def _g():
                y = jnp.dot(comm_ref[slot, :, :], w_bf[:, :],
                            preferred_element_type=jnp.float32)
                out_ref[pl.ds(origin(kind, c) * m_per, m_per), :] = \
                    _gelu_f32(y)
            if p is None:
                _g()
            else:
                @pl.when(p)
                def _():
                    _g()

        def col_step(d):
            for kind in ("bel", "abv"):
                c = _cidx(kind, d)
                p = pred_c(c)
                isem = c - 1 if kind == "bel" else 3 + (c - 4)
                for h in (0, 1):
                    _wait(_rdma(c, c, left, csend.at[0, h],
                                crecv.at[isem, h], h), p)
                    if d < 3:
                        nc = c + 1
                        if kind == "bel":
                            fp = jnp.logical_and(p, has_up)
                            _start(_rdma(c, nc, up, csend.at[2 + (d - 1), h],
                                         crecv.at[nc - 1, h], h), fp)
                        else:
                            fp = jnp.logical_and(p, has_dn)
                            _start(_rdma(c, nc, down,
                                         csend.at[4 + (d - 1), h],
                                         crecv.at[3 + (nc - 4), h], h), fp)
                    _start(_rdma(c, 7 + c, right, ssend.at[c, h],
                                 srecv.at[c, h], h), p)
                    _start(_rdma(c, 14 + c, left, ssend.at[c, 2 + h],
                                 srecv.at[c, 2 + h], h), p)
                _gemm_block(c, "me", c, p)

        def sq_step(c):
            p = pred_c(c)
            _wait(_rdma(7 + c, 7 + c, left, ssend.at[c, 0],
                        srecv.at[c, 0], 0), p)
            _start(_rdma(7 + c, 21 + c, right, ssend.at[c, 4],
                         srecv.at[c, 4], 0), p)
            _wait(_rdma(14 + c, 14 + c, left, ssend.at[c, 3],
                        srecv.at[c, 3], 1), p)
            _start(_rdma(14 + c, 21 + c, left, ssend.at[c, 5],
                         srecv.at[c, 5], 1), p)
            _wait(_rdma(7 + c, 7 + c, left, ssend.at[c, 1],
                        srecv.at[c, 1], 1), p)
            _wait(_rdma(14 + c, 14 + c, left, ssend.at[c, 2],
                        srecv.at[c, 2], 0), p)
            _gemm_block(7 + c, "L", c, p)
            _gemm_block(14 + c, "R", c, p)

        def diag_step(c):
            p = pred_c(c)
            _wait(_rdma(21 + c, 21 + c, left, ssend.at[c, 4],
                        srecv.at[c, 4], 0), p)
            _wait(_rdma(21 + c, 21 + c, left, ssend.at[c, 5],
                        srecv.at[c, 5], 1), p)
            _gemm_block(21 + c, "D", c, p)

        col_step(1)
        sq_step(0)
        col_step(2)
        sq_step(1)
        sq_step(4)
        col_step(3)
        sq_step(2)
        sq_step(5)
        diag_step(0)
        sq_step(3)
        sq_step(6)
        diag_step(1)
        diag_step(4)
        diag_step(2)
        diag_step(5)
        diag_step(3)
        diag_step(6)

        for rdma, pred in sends:
            if pred is None:
                rdma.wait_send()
            else:
                @pl.when(pred)
                def _():
                    rdma.wait_send()

    return pl.pallas_call(
        body,
        out_shape=jax.ShapeDtypeStruct((N_DEV * m_per, n_per), jnp.float32),
        in_specs=[
            pl.BlockSpec(memory_space=pltpu.VMEM),
            pl.BlockSpec(memory_space=pltpu.VMEM),
        ],
        out_specs=pl.BlockSpec(memory_space=pltpu.VMEM),
        scratch_shapes=[
            pltpu.VMEM((28, m_per, k), jnp.bfloat16),
            pltpu.VMEM((k, n_per), jnp.bfloat16),
            pltpu.SemaphoreType.DMA((6, 2)),
            pltpu.SemaphoreType.DMA((6, 2)),
            pltpu.SemaphoreType.DMA((N_C, 6)),
            pltpu.SemaphoreType.DMA((N_C, 6)),
        ],
        compiler_params=pltpu.CompilerParams(collective_id=0),
    )(x, w_mat)
